# baseline (speedup 1.0000x reference)
import jax
import jax.numpy as jnp
from jax import lax
from jax.experimental import pallas as pl
from jax.experimental.pallas import tpu as pltpu
from jax.experimental.pallas import tpu_sc as plsc

B, C, D, H, W = 4, 128, 2, 200, 176
S = D * H * W
Q = B * S
N = 40000
NT = 32
KW = 256
KWL = 8
NWG = Q // KW
WPB = S // KW
NWJ = (NWG + NT - 1) // NT
TQL = NWJ * KW
GCAP = TQL + KW
GR = 32
RD = 8
RING = RD * GR
SCAN = 512
NCH = 78
TAIL = N - NCH * SCAN
IMAX = 0x7FFFFFFF


def _body(feat, bidx, zidx, yidx, xidx, out,
          wid, bbA, zbA, ybA, xbA, bbB, zbB, ybB, xbB, sc64,
          jlist, idxlist, ring, outTA, outTB, starts,
          gsem, osemA, osemB, ssemA, ssemB):
    t = lax.axis_index("c") * 16 + lax.axis_index("s")
    nw_t = jnp.where(t < NWG - (NWJ - 1) * NT, NWJ, NWJ - 1)
    iota = lax.iota(jnp.int32, 16)
    z16f = jnp.zeros((16,), jnp.float32)
    z16i = jnp.zeros((16,), jnp.int32)
    cvecs = [c8 * 16 + iota for c8 in range(8)]

    def zwid(k, _):
        wid[pl.ds(k * 16, 16)] = z16i
        return 0
    lax.fori_loop(0, TQL // 16, zwid, 0)
    sc64[pl.ds(16, 16)] = jnp.full((16,), -1, jnp.int32)
    sc64[pl.ds(48, 16)] = jnp.full((16,), -1, jnp.int32)

    def zot(k, _):
        c = k // (KW // 16)
        o = (k % (KW // 16)) * 16
        outTA[c, pl.ds(o, 16)] = z16f
        outTB[c, pl.ds(o, 16)] = z16f
        return 0
    lax.fori_loop(0, C * (KW // 16), zot, 0)

    bufsA = (bbA, zbA, ybA, xbA)
    bufsB = (bbB, zbB, ybB, xbB)
    srcs = (bidx, zidx, yidx, xidx)

    def win_key(bufs, k2, half, n_vec):
        base = k2 * 32 + half * 16
        bv = bufs[0][pl.ds(base, 16)]
        zv = bufs[1][pl.ds(base, 16)]
        yv = bufs[2][pl.ds(base, 16)]
        xv = bufs[3][pl.ds(base, 16)]
        qv = ((bv * D + zv) * H + yv) * W + xv
        wk = qv >> KWL
        inr = (wk & (NT - 1)) == t
        jloc = ((wk >> 5) << KWL) | (qv & (KW - 1))
        return jnp.where(inr, (jloc << 16) | n_vec, IMAX)

    def scan_pair(bufs, off, k2):
        n0 = (off + k2 * 32) + iota
        n1 = n0 + 16
        key0 = win_key(bufs, k2, 0, n0)
        key1 = win_key(bufs, k2, 1, n1)
        sk0, sv0 = plsc.sort_key_val(key0, n0 + 1)
        sk1, sv1 = plsc.sort_key_val(key1, n1 + 1)
        sc64[pl.ds(0, 16)] = sk0
        sc64[pl.ds(32, 16)] = sk1
        nk0 = plsc.load_gather(sc64, [iota + 1])
        nk1 = plsc.load_gather(sc64, [iota + 33])
        q0 = sk0 >> 16
        q1 = sk1 >> 16
        keep0 = (q0 < TQL) & (q0 != (nk0 >> 16))
        keep1 = (q1 < TQL) & (q1 != (nk1 >> 16))
        plsc.store_scatter(wid, [q0], sv0, mask=keep0)
        plsc.store_scatter(wid, [q1], sv1, mask=keep1)

    def issue4(off, size, bufs, sem):
        for src, dst in zip(srcs, bufs):
            pltpu.async_copy(src.at[pl.ds(off, size)],
                             dst.at[pl.ds(0, size)], sem)

    def wait4(size, bufs, sem):
        for src, dst in zip(srcs, bufs):
            pltpu.make_async_copy(src.at[pl.ds(0, size)],
                                  dst.at[pl.ds(0, size)], sem).wait()

    def scan_chunk_of(bufs, off):
        def vb(k2, _):
            scan_pair(bufs, off, k2)
            return 0
        lax.fori_loop(0, SCAN // 32, vb, 0)

    with jax.named_scope("p1_scan"):
        issue4(0, SCAN, bufsA, ssemA)

        def pchunk(i, _):
            offA = pl.multiple_of(2 * i * SCAN, SCAN)
            offB = pl.multiple_of((2 * i + 1) * SCAN, SCAN)
            issue4(offB, SCAN, bufsB, ssemB)
            wait4(SCAN, bufsA, ssemA)
            scan_chunk_of(bufsA, offA)

            @pl.when(i < NCH // 2 - 1)
            def _():
                issue4(pl.multiple_of((2 * i + 2) * SCAN, SCAN), SCAN,
                       bufsA, ssemA)
            wait4(SCAN, bufsB, ssemB)
            scan_chunk_of(bufsB, offB)
            return 0
        lax.fori_loop(0, NCH // 2, pchunk, 0)

        toff = NCH * SCAN
        issue4(toff, TAIL, bufsA, ssemA)
        wait4(TAIL, bufsA, ssemA)

        def tail_body(k2, _):
            scan_pair(bufsA, toff, k2)
            return 0
        lax.fori_loop(0, TAIL // 32, tail_body, 0)

    with jax.named_scope("p2a_compact"):
        starts[0] = jnp.int32(0)

        def scanw(lw, cnt):
            def sck(k, c):
                wv = wid[pl.ds(lw * KW + k * 16, 16)]
                m = wv > 0
                plsc.store_compressed(jlist.at[pl.ds(c, 16)], k * 16 + iota,
                                      mask=m)
                plsc.store_compressed(idxlist.at[pl.ds(c, 16)], wv - 1,
                                      mask=m)
                return c + jnp.max(plsc.all_reduce_population_count(m))
            cnt = lax.fori_loop(0, KW // 16, sck, cnt)
            starts[lw + 1] = cnt
            return cnt
        U = lax.fori_loop(0, nw_t, scanw, jnp.int32(0))

        def phantom(lw, _):
            starts[lw + 1] = U
            return 0
        lax.fori_loop(nw_t, NWJ + 1, phantom, 0)

        def padi(k, _):
            idxlist[pl.ds(U + k * 16, 16)] = z16i
            return 0
        lax.fori_loop(0, KW // 16, padi, 0)
        nd = (U + GR - 1) // GR

    def process_window(lw, outT, osem, dI, dR):
        live = lw < nw_t
        start_w = starts[jnp.minimum(lw, NWJ)]
        end_w = starts[jnp.minimum(lw, NWJ) + 1]

        @pl.when((lw >= 2) & live)
        def _():
            pltpu.make_async_copy(
                outT, out.at[0, :, pl.ds(0, KW)], osem).wait()
            s_p = starts[lw - 2]
            e_p = starts[lw - 1]

            def rz(u, _):
                col = jlist[pl.ds(u, 16)][0]
                bc = jnp.broadcast_to(col, (16,))
                for c8 in range(8):
                    plsc.store_scatter(outT, [cvecs[c8], bc], z16f)
                return 0
            lax.fori_loop(s_p, e_p, rz, 0)

        def icond(d):
            return ((d < nd) & (d * GR < end_w + (RD // 2) * GR)
                    & ((d < RD) | ((d - (RD - 1)) * GR <= start_w)))

        def ibody(d):
            slot = (d & (RD - 1)) * GR
            pltpu.async_copy(feat.at[idxlist.at[pl.ds(d * GR, GR)]],
                             ring.at[pl.ds(slot, GR)], gsem)
            return d + 1
        dI = lax.while_loop(icond, ibody, dI)

        need = (end_w + GR - 1) // GR

        def dbody(d):
            pltpu.make_async_copy(feat.at[idxlist.at[pl.ds(0, GR)]],
                                  ring.at[pl.ds(0, GR)], gsem).wait()
            return d + 1
        dR = lax.while_loop(lambda d: d < need, dbody, dR)

        def sg(u, _):
            col = jlist[pl.ds(u, 16)][0]
            bc = jnp.broadcast_to(col, (16,))
            r = u & (RING - 1)
            for c8 in range(8):
                v = ring[r, pl.ds(c8 * 16, 16)]
                plsc.store_scatter(outT, [cvecs[c8], bc], v)
            return 0
        lax.fori_loop(start_w, end_w, sg, 0)

        @pl.when(live)
        def _():
            gw = t + NT * lw
            b = gw // WPB
            s0 = pl.multiple_of((gw % WPB) * KW, KW)
            pltpu.async_copy(outT, out.at[b, :, pl.ds(s0, KW)], osem)
        return dI, dR

    with jax.named_scope("p2b_windows"):
        def outer(i, carry):
            dI, dR = carry
            dI, dR = process_window(2 * i, outTA, osemA, dI, dR)
            dI, dR = process_window(2 * i + 1, outTB, osemB, dI, dR)
            return (dI, dR)
        lax.fori_loop(0, (NWJ + 1) // 2, outer,
                      (jnp.int32(0), jnp.int32(0)))

    pltpu.make_async_copy(outTA, out.at[0, :, pl.ds(0, KW)], osemA).wait()
    pltpu.make_async_copy(outTB, out.at[0, :, pl.ds(0, KW)], osemB).wait()


@jax.jit
def kernel(features, batch_idx, z_idx, y_idx, x_idx):
    mesh = plsc.VectorSubcoreMesh(core_axis_name="c", subcore_axis_name="s")
    run = pl.kernel(
        _body,
        out_type=jax.ShapeDtypeStruct((B, C, S), jnp.float32),
        mesh=mesh,
        compiler_params=pltpu.CompilerParams(
            use_tc_tiling_on_sc=True, needs_layout_passes=False),
        scratch_types=[
            pltpu.VMEM((TQL,), jnp.int32),
            pltpu.VMEM((SCAN,), jnp.int32),
            pltpu.VMEM((SCAN,), jnp.int32),
            pltpu.VMEM((SCAN,), jnp.int32),
            pltpu.VMEM((SCAN,), jnp.int32),
            pltpu.VMEM((SCAN,), jnp.int32),
            pltpu.VMEM((SCAN,), jnp.int32),
            pltpu.VMEM((SCAN,), jnp.int32),
            pltpu.VMEM((SCAN,), jnp.int32),
            pltpu.VMEM((64,), jnp.int32),
            pltpu.VMEM((GCAP,), jnp.int32),
            pltpu.VMEM((GCAP,), jnp.int32),
            pltpu.VMEM((RING, C), jnp.float32),
            pltpu.VMEM((C, KW), jnp.float32),
            pltpu.VMEM((C, KW), jnp.float32),
            pltpu.SMEM((NWJ + 2,), jnp.int32),
            pltpu.SemaphoreType.DMA,
            pltpu.SemaphoreType.DMA,
            pltpu.SemaphoreType.DMA,
            pltpu.SemaphoreType.DMA,
            pltpu.SemaphoreType.DMA,
        ],
    )
    dense = run(features, batch_idx, z_idx, y_idx, x_idx)
    return dense.reshape(B, C, D, H, W)
